# baseline (speedup 1.0000x reference)
import functools

import jax
import jax.numpy as jnp
from jax import lax
from jax.experimental import pallas as pl
from jax.experimental.pallas import tpu as pltpu
from jax.experimental.pallas import tpu_sc as plsc

_LANES = 16


@functools.lru_cache(maxsize=None)
def _make_sc_gather(B, n, d, CH, NBUF):
    info = plsc.get_sparse_core_info()
    NC, NS = info.num_cores, info.num_subcores
    NW = NC * NS
    R = (B * n) // NW
    assert (B * n) % NW == 0 and n % R == 0 or R % n == 0
    assert R % CH == 0 and R % _LANES == 0
    NCH = R // CH
    assert NCH % NBUF == 0

    mesh = plsc.VectorSubcoreMesh(core_axis_name="c", subcore_axis_name="s")
    out_type = jax.ShapeDtypeStruct((B * n, d), jnp.float32)
    scratch = [pltpu.VMEM((R,), jnp.int32)] if R <= n else \
        [pltpu.VMEM((n,), jnp.int32)]
    scratch += [pltpu.VMEM((CH, d), jnp.float32) for _ in range(NBUF)]
    scratch += [pltpu.SemaphoreType.DMA for _ in range(2 * NBUF)]

    @functools.partial(pl.kernel, mesh=mesh, out_type=out_type,
                       scratch_types=scratch)
    def sc_kernel(emb, order, out, idx, *rest):
        bufs = rest[:NBUF]
        gsem = rest[NBUF:2 * NBUF]
        ssem = rest[2 * NBUF:]
        wid = lax.axis_index("s") * NC + lax.axis_index("c")

        n_inner = max(1, R // n)
        RR = min(R, n)
        for kk in range(n_inner):
            row0 = wid * R + kk * RR
            b = row0 // n
            i0 = row0 - b * n
            base = b * n

            pltpu.sync_copy(order.at[pl.ds(i0, RR)], idx.at[pl.ds(0, RR)])
            for j in range(RR // _LANES):
                sl = pl.ds(j * _LANES, _LANES)
                idx[sl] = idx[sl] + base

            def gather_start(c, s):
                pltpu.async_copy(emb.at[idx.at[pl.ds(c * CH, CH)]],
                                 bufs[s], gsem[s])

            def gather_wait(c, s):
                pltpu.make_async_copy(emb.at[idx.at[pl.ds(c * CH, CH)]],
                                      bufs[s], gsem[s]).wait()

            def store_start(c, s):
                pltpu.async_copy(bufs[s], out.at[pl.ds(row0 + c * CH, CH)],
                                 ssem[s])

            def store_wait(c, s):
                pltpu.make_async_copy(bufs[s],
                                      out.at[pl.ds(row0 + c * CH, CH)],
                                      ssem[s]).wait()

            NCHi = RR // CH
            for s in range(NBUF):
                gather_start(s, s)

            def body(i, _):
                for s in range(NBUF):
                    c = i * NBUF + s
                    gather_wait(c, s)
                    store_start(c, s)
                for s in range(NBUF):
                    c2 = (i + 1) * NBUF + s

                    def refill(s=s, c2=c2):
                        store_wait(c2 - NBUF, s)
                        gather_start(c2, s)

                    pl.when(c2 < NCHi)(refill)
                return 0

            lax.fori_loop(0, NCHi // NBUF, body, 0)

            for s in range(NBUF):
                store_wait(NCHi - NBUF + s, s)

    return sc_kernel


@functools.lru_cache(maxsize=None)
def _make_sc_gather2(B, n, d, CHG, QPS, NBUF):
    info = plsc.get_sparse_core_info()
    NC, NS = info.num_cores, info.num_subcores
    NW = NC * NS
    CHS = CHG * QPS
    R = (B * n) // NW
    assert (B * n) % NW == 0 and (n % R == 0 or R % n == 0)
    RR = min(R, n)
    assert RR % CHS == 0 and RR % _LANES == 0
    NCH = RR // CHS
    assert NCH % NBUF == 0

    mesh = plsc.VectorSubcoreMesh(core_axis_name="c", subcore_axis_name="s")
    out_type = jax.ShapeDtypeStruct((B * n, d), jnp.float32)
    scratch = [pltpu.VMEM((RR,), jnp.int32)]
    scratch += [pltpu.VMEM((CHS, d), jnp.float32) for _ in range(NBUF)]
    scratch += [pltpu.SemaphoreType.DMA for _ in range(NBUF * QPS)]
    scratch += [pltpu.SemaphoreType.DMA for _ in range(NBUF)]

    @functools.partial(pl.kernel, mesh=mesh, out_type=out_type,
                       scratch_types=scratch)
    def sc_kernel(emb, order, out, idx, *rest):
        bufs = rest[:NBUF]
        gsem = rest[NBUF:NBUF + NBUF * QPS]
        ssem = rest[NBUF + NBUF * QPS:]
        wid = lax.axis_index("s") * NC + lax.axis_index("c")

        n_inner = max(1, R // n)
        for kk in range(n_inner):
            row0 = wid * R + kk * RR
            b = row0 // n
            i0 = row0 - b * n
            base = b * n

            pltpu.sync_copy(order.at[pl.ds(i0, RR)], idx)
            for j in range(RR // _LANES):
                sl = pl.ds(j * _LANES, _LANES)
                idx[sl] = idx[sl] + base

            def gather_start(c, s, q):
                pltpu.async_copy(
                    emb.at[idx.at[pl.ds(c * CHS + q * CHG, CHG)]],
                    bufs[s].at[pl.ds(q * CHG, CHG)], gsem[s * QPS + q])

            def gather_wait(c, s, q):
                pltpu.make_async_copy(
                    emb.at[idx.at[pl.ds(c * CHS + q * CHG, CHG)]],
                    bufs[s].at[pl.ds(q * CHG, CHG)],
                    gsem[s * QPS + q]).wait()

            def store_start(c, s):
                pltpu.async_copy(bufs[s], out.at[pl.ds(row0 + c * CHS, CHS)],
                                 ssem[s])

            def store_wait(c, s):
                pltpu.make_async_copy(bufs[s],
                                      out.at[pl.ds(row0 + c * CHS, CHS)],
                                      ssem[s]).wait()

            for s in range(NBUF):
                for q in range(QPS):
                    gather_start(s, s, q)

            def body(i, _):
                for s in range(NBUF):
                    c = i * NBUF + s
                    for q in range(QPS):
                        gather_wait(c, s, q)
                    store_start(c, s)
                for s in range(NBUF):
                    c2 = (i + 1) * NBUF + s

                    def refill(s=s, c2=c2):
                        store_wait(c2 - NBUF, s)
                        for q in range(QPS):
                            gather_start(c2, s, q)

                    pl.when(c2 < NCH)(refill)
                return 0

            lax.fori_loop(0, NCH // NBUF, body, 0)

            for s in range(NBUF):
                store_wait(NCH - NBUF + s, s)

    return sc_kernel


@functools.lru_cache(maxsize=None)
def _make_tc_gather(B, n, d, NQ=16):
    def body(ord_ref, emb_ref, out_ref, *sems):
        def start(c, s):
            src = ord_ref[c] * d
            pltpu.make_async_copy(
                emb_ref.at[:, pl.ds(src, d)],
                out_ref.at[:, pl.ds(c * d, d)],
                sems[s]).start()

        def wait(c, s):
            pltpu.make_async_copy(
                emb_ref.at[:, pl.ds(ord_ref[c] * d, d)],
                out_ref.at[:, pl.ds(c * d, d)],
                sems[s]).wait()

        for s in range(NQ):
            start(s, s)

        def loop(i, _):
            for s in range(NQ):
                c = i * NQ + s
                wait(c, s)
                c2 = (i + 1) * NQ + s

                def refill(s=s, c2=c2):
                    start(c2, s)

                pl.when(c2 < n)(refill)
            return 0

        lax.fori_loop(0, n // NQ, loop, 0)

    assert n % NQ == 0
    return pl.pallas_call(
        body,
        grid_spec=pltpu.PrefetchScalarGridSpec(
            num_scalar_prefetch=1,
            grid=(),
            in_specs=[pl.BlockSpec(memory_space=pl.ANY)],
            out_specs=pl.BlockSpec(memory_space=pl.ANY),
            scratch_shapes=[pltpu.SemaphoreType.DMA] * NQ,
        ),
        out_shape=jax.ShapeDtypeStruct((B, n * d), jnp.float32),
    )


_B_SC = 32


def kernel(embeddings, order):
    B, n, d = embeddings.shape
    order_i = order.astype(jnp.int32)
    parts = []
    if _B_SC > 0:
        sc = _make_sc_gather2(_B_SC, n, d, 16, 4, 2)
        sc_out = sc(embeddings[:_B_SC].reshape(_B_SC * n, d), order_i)
        parts.append(sc_out.reshape(_B_SC, n, d))
    if _B_SC < B:
        B_tc = B - _B_SC
        tc = _make_tc_gather(B_tc, n, d, 16)
        tc_out = tc(order_i, embeddings[_B_SC:].reshape(B_tc, n * d))
        parts.append(tc_out.reshape(B_tc, n, d))
    out = parts[0] if len(parts) == 1 else jnp.concatenate(parts, axis=0)
    return (out, None)


# speedup vs baseline: 1.0107x; 1.0107x over previous
import functools

import jax
import jax.numpy as jnp
from jax import lax
from jax.experimental import pallas as pl
from jax.experimental.pallas import tpu as pltpu
from jax.experimental.pallas import tpu_sc as plsc

_LANES = 16


@functools.lru_cache(maxsize=None)
def _make_sc_gather(B, n, d, CHG, QPS, NBUF):
    info = plsc.get_sparse_core_info()
    NC, NS = info.num_cores, info.num_subcores
    NW = NC * NS
    CHS = CHG * QPS
    R = (B * n) // NW
    assert (B * n) % NW == 0 and (n % R == 0 or R % n == 0)
    RR = min(R, n)
    assert RR % CHS == 0 and RR % _LANES == 0
    NCH = RR // CHS
    assert NCH % NBUF == 0

    mesh = plsc.VectorSubcoreMesh(core_axis_name="c", subcore_axis_name="s")
    out_type = jax.ShapeDtypeStruct((B * n, d), jnp.float32)
    scratch = [pltpu.VMEM((RR,), jnp.int32)]
    scratch += [pltpu.VMEM((CHS, d), jnp.float32) for _ in range(NBUF)]
    scratch += [pltpu.SemaphoreType.DMA for _ in range(NBUF * QPS)]
    scratch += [pltpu.SemaphoreType.DMA for _ in range(NBUF)]

    @functools.partial(pl.kernel, mesh=mesh, out_type=out_type,
                       scratch_types=scratch)
    def sc_kernel(emb, order, out, idx, *rest):
        bufs = rest[:NBUF]
        gsem = rest[NBUF:NBUF + NBUF * QPS]
        ssem = rest[NBUF + NBUF * QPS:]
        wid = lax.axis_index("s") * NC + lax.axis_index("c")

        n_inner = max(1, R // n)
        for kk in range(n_inner):
            row0 = wid * R + kk * RR
            b = row0 // n
            i0 = row0 - b * n
            base = b * n

            pltpu.sync_copy(order.at[pl.ds(i0, RR)], idx)
            for j in range(RR // _LANES):
                sl = pl.ds(j * _LANES, _LANES)
                idx[sl] = idx[sl] + base

            def gather_start(c, s, q):
                pltpu.async_copy(
                    emb.at[idx.at[pl.ds(c * CHS + q * CHG, CHG)]],
                    bufs[s].at[pl.ds(q * CHG, CHG)], gsem[s * QPS + q])

            def gather_wait(c, s, q):
                pltpu.make_async_copy(
                    emb.at[idx.at[pl.ds(c * CHS + q * CHG, CHG)]],
                    bufs[s].at[pl.ds(q * CHG, CHG)],
                    gsem[s * QPS + q]).wait()

            def store_start(c, s):
                pltpu.async_copy(bufs[s], out.at[pl.ds(row0 + c * CHS, CHS)],
                                 ssem[s])

            def store_wait(c, s):
                pltpu.make_async_copy(bufs[s],
                                      out.at[pl.ds(row0 + c * CHS, CHS)],
                                      ssem[s]).wait()

            for s in range(NBUF):
                for q in range(QPS):
                    gather_start(s, s, q)

            def body(i, _):
                for s in range(NBUF):
                    c = i * NBUF + s
                    for q in range(QPS):
                        gather_wait(c, s, q)
                    store_start(c, s)
                for s in range(NBUF):
                    c2 = (i + 1) * NBUF + s

                    def refill(s=s, c2=c2):
                        store_wait(c2 - NBUF, s)
                        for q in range(QPS):
                            gather_start(c2, s, q)

                    pl.when(c2 < NCH)(refill)
                return 0

            lax.fori_loop(0, NCH // NBUF, body, 0)

            for s in range(NBUF):
                store_wait(NCH - NBUF + s, s)

    return sc_kernel


def kernel(embeddings, order):
    B, n, d = embeddings.shape
    order_i = order.astype(jnp.int32)
    f = _make_sc_gather(B, n, d, 16, 2, 4)
    out = f(embeddings.reshape(B * n, d), order_i)
    return (out.reshape(B, n, d), None)
